# Initial kernel scaffold; baseline (speedup 1.0000x reference)
import functools

import jax
import jax.numpy as jnp
from jax import lax
from jax.experimental import pallas as pl
from jax.experimental.pallas import tpu as pltpu
from jax.experimental.pallas import tpu_sc as plsc

N = 10000
E = 320000
DIN = 128
DE = 16
H = 64
G = 16
OUT = 1

NSC = 2
NTILE = 16
NW = NSC * NTILE
CHUNK = 128
NCHUNKS = E // CHUNK
ITERS = -(-NCHUNKS // NW)
ROWS = N // NTILE
EB = 2560

_LANES = H // 16


def _node_prep_body(x_ref, wnT_ref, bn_ref, wmiT_ref, wmjT_ref,
                    xh_ref, a_ref, b_ref):
    xh = jnp.maximum(x_ref[...] @ wnT_ref[...] + bn_ref[...], 0.0)
    xh_ref[...] = xh
    a_ref[...] = xh @ wmiT_ref[...]
    b_ref[...] = xh @ wmjT_ref[...]


def _edge_prep_body(ea_ref, weT_ref, be_ref, wmeT_ref, bM_ref, c_ref):
    eh = jnp.maximum(ea_ref[...] @ weT_ref[...] + be_ref[...], 0.0)
    c_ref[...] = eh @ wmeT_ref[...] + bM_ref[...]


def _sc_msg_body(a_hbm, b_hbm, c_hbm, dst_hbm, src_hbm, zs_hbm, zc_hbm,
                 s_out, cnt_out,
                 dst_v, src_v, a_v, b_v, c_v, ones_v, s_sh, cnt_sh,
                 sem_a, sem_b):
    cid = lax.axis_index("c")
    tid = lax.axis_index("s")
    w = cid * NTILE + tid

    pltpu.sync_copy(zs_hbm.at[pl.ds(tid * ROWS, ROWS)],
                    s_sh.at[pl.ds(tid * ROWS, ROWS)])
    pltpu.sync_copy(zc_hbm.at[pl.ds(tid * ROWS, ROWS)],
                    cnt_sh.at[pl.ds(tid * ROWS, ROWS)])

    def _fill(i, _):
        ones_v[i, :] = jnp.ones((16,), jnp.float32)
        return 0
    lax.fori_loop(0, CHUNK, _fill, 0)

    plsc.subcore_barrier()

    def body(i, _):
        ch = w + i * NW

        @pl.when(ch < NCHUNKS)
        def _():
            base = ch * CHUNK
            pltpu.sync_copy(dst_hbm.at[pl.ds(base, CHUNK)], dst_v)
            pltpu.sync_copy(src_hbm.at[pl.ds(base, CHUNK)], src_v)
            pltpu.sync_copy(c_hbm.at[pl.ds(base, CHUNK)], c_v)
            pltpu.async_copy(a_hbm.at[dst_v], a_v, sem_a).wait()
            pltpu.async_copy(b_hbm.at[src_v], b_v, sem_b).wait()

            def row(r, _):
                for j in range(_LANES):
                    sl = pl.ds(j * 16, 16)
                    c_v[r, sl] = jnp.maximum(
                        a_v[r, sl] + b_v[r, sl] + c_v[r, sl], 0.0)
                return 0
            lax.fori_loop(0, CHUNK, row, 0)

            pltpu.sync_copy(c_v, s_sh.at[dst_v], add=True)
            pltpu.sync_copy(ones_v, cnt_sh.at[dst_v], add=True)
        return 0

    lax.fori_loop(0, ITERS, body, 0)

    plsc.subcore_barrier()

    pltpu.sync_copy(s_sh.at[pl.ds(tid * ROWS, ROWS)],
                    s_out.at[pl.ds(cid * N + tid * ROWS, ROWS)])
    pltpu.sync_copy(cnt_sh.at[pl.ds(tid * ROWS, ROWS)],
                    cnt_out.at[pl.ds(cid * N + tid * ROWS, ROWS)])


def _update_pool_body(xh_ref, s_ref, cnt_ref, bv_ref,
                      wu1T_ref, wu2T_ref, bU_ref, wrT_ref, br_ref, out_ref):
    s = s_ref[:N, :] + s_ref[N:, :]
    cnt = cnt_ref[:N, 0:1] + cnt_ref[N:, 0:1]
    aggr = s / jnp.maximum(cnt, 1.0)
    upd = jnp.maximum(
        xh_ref[...] @ wu1T_ref[...] + aggr @ wu2T_ref[...] + bU_ref[...], 0.0)
    gids = lax.broadcasted_iota(jnp.int32, (N, G), 1)
    oh = (bv_ref[...] == gids).astype(jnp.float32)
    ps = lax.dot_general(oh, upd, (((0,), (0,)), ((), ())))
    pc = jnp.sum(oh, axis=0)[:, None]
    pooled = ps / jnp.maximum(pc, 1.0)
    out_ref[...] = pooled @ wrT_ref[...] + br_ref[...]


def _full(shape):
    return pl.BlockSpec(shape, lambda *a: tuple(0 for _ in shape))


def kernel(x, edge_index, edge_attr, batch_vec, Wn, bn, We, be, WM, bM,
           WU, bU, Wr, br):
    f32 = jnp.float32
    WnT = Wn.T
    WMiT = WM[:, :H].T
    WMjT = WM[:, H:2 * H].T
    WMeT = WM[:, 2 * H:].T
    WU1T = WU[:, :H].T
    WU2T = WU[:, H:].T
    WrT = Wr.T
    bn2 = bn.reshape(1, H)
    be2 = be.reshape(1, H)
    bM2 = bM.reshape(1, H)
    bU2 = bU.reshape(1, H)
    br2 = br.reshape(1, OUT)
    src = edge_index[0]
    dst = edge_index[1]
    bv2 = batch_vec.reshape(N, 1)

    xh, a_nodes, b_nodes = pl.pallas_call(
        _node_prep_body,
        out_shape=[jax.ShapeDtypeStruct((N, H), f32)] * 3,
        in_specs=[_full((N, DIN)), _full((DIN, H)), _full((1, H)),
                  _full((H, H)), _full((H, H))],
        out_specs=[_full((N, H))] * 3,
    )(x, WnT, bn2, WMiT, WMjT)

    c_edges = pl.pallas_call(
        _edge_prep_body,
        grid=(E // EB,),
        out_shape=jax.ShapeDtypeStruct((E, H), f32),
        in_specs=[pl.BlockSpec((EB, DE), lambda i: (i, 0)),
                  _full((DE, H)), _full((1, H)), _full((H, H)),
                  _full((1, H))],
        out_specs=pl.BlockSpec((EB, H), lambda i: (i, 0)),
    )(edge_attr, We.T, be2, WMeT, bM2)

    mesh = plsc.VectorSubcoreMesh(core_axis_name="c", subcore_axis_name="s",
                                  num_cores=NSC, num_subcores=NTILE)
    sc_call = pl.kernel(
        _sc_msg_body,
        out_type=[jax.ShapeDtypeStruct((NSC * N, H), f32),
                  jax.ShapeDtypeStruct((NSC * N, 16), f32)],
        mesh=mesh,
        scratch_types=[
            pltpu.VMEM((CHUNK,), jnp.int32),
            pltpu.VMEM((CHUNK,), jnp.int32),
            pltpu.VMEM((CHUNK, H), f32),
            pltpu.VMEM((CHUNK, H), f32),
            pltpu.VMEM((CHUNK, H), f32),
            pltpu.VMEM((CHUNK, 16), f32),
            pltpu.VMEM_SHARED((N, H), f32),
            pltpu.VMEM_SHARED((N, 16), f32),
            pltpu.SemaphoreType.DMA,
            pltpu.SemaphoreType.DMA,
        ],
    )
    zs = jnp.zeros((N, H), f32)
    zc = jnp.zeros((N, 16), f32)
    s_par, cnt_par = sc_call(a_nodes, b_nodes, c_edges, dst, src, zs, zc)

    out = pl.pallas_call(
        _update_pool_body,
        out_shape=jax.ShapeDtypeStruct((G, OUT), f32),
        in_specs=[_full((N, H)), _full((NSC * N, H)), _full((NSC * N, 16)),
                  _full((N, 1)), _full((H, H)), _full((H, H)), _full((1, H)),
                  _full((H, OUT)), _full((1, OUT))],
        out_specs=_full((G, OUT)),
    )(xh, s_par, cnt_par, bv2, WU1T, WU2T, bU2, WrT, br2)
    return out


# speedup vs baseline: 3.6890x; 3.6890x over previous
import functools

import jax
import jax.numpy as jnp
from jax import lax
from jax.experimental import pallas as pl
from jax.experimental.pallas import tpu as pltpu
from jax.experimental.pallas import tpu_sc as plsc

N = 10000
E = 320000
DIN = 128
DE = 16
H = 64
G = 16
OUT = 1

NSC = 2
NTILE = 16
NW = NSC * NTILE
CHUNK = 128
NCHUNKS = E // CHUNK
ITERS = -(-NCHUNKS // NW)
NPAD = 10240
ROWS = NPAD // NTILE
EB = 2560

_LANES = H // 16


def _node_prep_body(x_ref, wnT_ref, bn_ref, wmiT_ref, wmjT_ref,
                    xh_ref, a_ref, b_ref):
    xh = jnp.maximum(x_ref[...] @ wnT_ref[...] + bn_ref[...], 0.0)
    xh_ref[...] = xh
    a_ref[...] = xh @ wmiT_ref[...]
    b_ref[...] = xh @ wmjT_ref[...]


def _edge_prep_body(ea_ref, weT_ref, be_ref, wmeT_ref, bM_ref, c_ref):
    eh = jnp.maximum(ea_ref[...] @ weT_ref[...] + be_ref[...], 0.0)
    c_ref[...] = eh @ wmeT_ref[...] + bM_ref[...]


def _sc_msg_body(a_hbm, b_hbm, c_hbm, dst_hbm, src_hbm, zs_hbm, zc_hbm,
                 s_out, cnt_out,
                 dst_v, src_v, a_v, b_v, c_v, ones_v, s_sh, cnt_sh,
                 sem_a, sem_b):
    cid = lax.axis_index("c")
    tid = lax.axis_index("s")
    w = cid * NTILE + tid

    pltpu.sync_copy(zs_hbm.at[pl.ds(tid * ROWS, ROWS)],
                    s_sh.at[pl.ds(tid * ROWS, ROWS)])
    pltpu.sync_copy(zc_hbm.at[pl.ds(tid * ROWS, ROWS)],
                    cnt_sh.at[pl.ds(tid * ROWS, ROWS)])

    def _fill(i, _):
        ones_v[i, :] = jnp.ones((16,), jnp.float32)
        return 0
    lax.fori_loop(0, CHUNK, _fill, 0)

    plsc.subcore_barrier()

    def body(i, _):
        ch = w + i * NW

        @pl.when(ch < NCHUNKS)
        def _():
            base = ch * CHUNK
            pltpu.sync_copy(dst_hbm.at[pl.ds(base, CHUNK)], dst_v)
            pltpu.sync_copy(src_hbm.at[pl.ds(base, CHUNK)], src_v)
            pltpu.sync_copy(c_hbm.at[pl.ds(base, CHUNK)], c_v)
            pltpu.async_copy(a_hbm.at[dst_v], a_v, sem_a).wait()
            pltpu.async_copy(b_hbm.at[src_v], b_v, sem_b).wait()

            def row(r, _):
                for j in range(_LANES):
                    sl = pl.ds(j * 16, 16)
                    c_v[r, sl] = jnp.maximum(
                        a_v[r, sl] + b_v[r, sl] + c_v[r, sl], 0.0)
                return 0
            lax.fori_loop(0, CHUNK, row, 0)

            pltpu.sync_copy(c_v, s_sh.at[dst_v], add=True)
            pltpu.sync_copy(ones_v, cnt_sh.at[dst_v], add=True)
        return 0

    lax.fori_loop(0, ITERS, body, 0)

    plsc.subcore_barrier()

    pltpu.sync_copy(s_sh.at[pl.ds(tid * ROWS, ROWS)],
                    s_out.at[pl.ds(cid * NPAD + tid * ROWS, ROWS)])
    pltpu.sync_copy(cnt_sh.at[pl.ds(tid * ROWS, ROWS)],
                    cnt_out.at[pl.ds(cid * NPAD + tid * ROWS, ROWS)])


def _update_pool_body(xh_ref, s_ref, cnt_ref, bv_ref,
                      wu1T_ref, wu2T_ref, bU_ref, wrT_ref, br_ref, out_ref):
    s = s_ref[:N, :] + s_ref[NPAD:NPAD + N, :]
    cnt = cnt_ref[:N, 0:1] + cnt_ref[NPAD:NPAD + N, 0:1]
    aggr = s / jnp.maximum(cnt, 1.0)
    upd = jnp.maximum(
        xh_ref[...] @ wu1T_ref[...] + aggr @ wu2T_ref[...] + bU_ref[...], 0.0)
    gids = lax.broadcasted_iota(jnp.int32, (N, G), 1)
    oh = (bv_ref[...] == gids).astype(jnp.float32)
    ps = lax.dot_general(oh, upd, (((0,), (0,)), ((), ())))
    pc = jnp.sum(oh, axis=0)[:, None]
    pooled = ps / jnp.maximum(pc, 1.0)
    out_ref[...] = pooled @ wrT_ref[...] + br_ref[...]


def _full(shape):
    return pl.BlockSpec(shape, lambda *a: tuple(0 for _ in shape))


def kernel(x, edge_index, edge_attr, batch_vec, Wn, bn, We, be, WM, bM,
           WU, bU, Wr, br):
    f32 = jnp.float32
    WnT = Wn.T
    WMiT = WM[:, :H].T
    WMjT = WM[:, H:2 * H].T
    WMeT = WM[:, 2 * H:].T
    WU1T = WU[:, :H].T
    WU2T = WU[:, H:].T
    WrT = Wr.T
    bn2 = bn.reshape(1, H)
    be2 = be.reshape(1, H)
    bM2 = bM.reshape(1, H)
    bU2 = bU.reshape(1, H)
    br2 = br.reshape(1, OUT)
    src = edge_index[0]
    dst = edge_index[1]
    bv2 = batch_vec.reshape(N, 1)

    xh, a_nodes, b_nodes = pl.pallas_call(
        _node_prep_body,
        out_shape=[jax.ShapeDtypeStruct((N, H), f32)] * 3,
        in_specs=[_full((N, DIN)), _full((DIN, H)), _full((1, H)),
                  _full((H, H)), _full((H, H))],
        out_specs=[_full((N, H))] * 3,
    )(x, WnT, bn2, WMiT, WMjT)

    c_edges = pl.pallas_call(
        _edge_prep_body,
        grid=(E // EB,),
        out_shape=jax.ShapeDtypeStruct((E, H), f32),
        in_specs=[pl.BlockSpec((EB, DE), lambda i: (i, 0)),
                  _full((DE, H)), _full((1, H)), _full((H, H)),
                  _full((1, H))],
        out_specs=pl.BlockSpec((EB, H), lambda i: (i, 0)),
    )(edge_attr, We.T, be2, WMeT, bM2)

    mesh = plsc.VectorSubcoreMesh(core_axis_name="c", subcore_axis_name="s",
                                  num_cores=NSC, num_subcores=NTILE)
    sc_call = pl.kernel(
        _sc_msg_body,
        out_type=[jax.ShapeDtypeStruct((NSC * NPAD, H), f32),
                  jax.ShapeDtypeStruct((NSC * NPAD, 16), f32)],
        mesh=mesh,
        compiler_params=pltpu.CompilerParams(use_tc_tiling_on_sc=False),
        scratch_types=[
            pltpu.VMEM((CHUNK,), jnp.int32),
            pltpu.VMEM((CHUNK,), jnp.int32),
            pltpu.VMEM((CHUNK, H), f32),
            pltpu.VMEM((CHUNK, H), f32),
            pltpu.VMEM((CHUNK, H), f32),
            pltpu.VMEM((CHUNK, 16), f32),
            pltpu.VMEM_SHARED((NPAD, H), f32),
            pltpu.VMEM_SHARED((NPAD, 16), f32),
            pltpu.SemaphoreType.DMA,
            pltpu.SemaphoreType.DMA,
        ],
    )
    zs = jnp.zeros((NPAD, H), f32)
    zc = jnp.zeros((NPAD, 16), f32)
    s_par, cnt_par = sc_call(a_nodes, b_nodes, c_edges, dst, src, zs, zc)

    out = pl.pallas_call(
        _update_pool_body,
        out_shape=jax.ShapeDtypeStruct((G, OUT), f32),
        in_specs=[_full((N, H)), _full((NSC * NPAD, H)), _full((NSC * NPAD, 16)),
                  _full((N, 1)), _full((H, H)), _full((H, H)), _full((1, H)),
                  _full((H, OUT)), _full((1, OUT))],
        out_specs=_full((G, OUT)),
    )(xh, s_par, cnt_par, bv2, WU1T, WU2T, bU2, WrT, br2)
    return out


# speedup vs baseline: 4.8086x; 1.3035x over previous
import functools

import jax
import jax.numpy as jnp
from jax import lax
from jax.experimental import pallas as pl
from jax.experimental.pallas import tpu as pltpu
from jax.experimental.pallas import tpu_sc as plsc

N = 10000
E = 320000
DIN = 128
DE = 16
H = 64
G = 16
OUT = 1

NSC = 2
NTILE = 16
NW = NSC * NTILE
CHUNK = 128
NCHUNKS = E // CHUNK
ITERS = -(-NCHUNKS // NW)
NPAD = 10240
ROWS = NPAD // NTILE
EB = 2560

_LANES = H // 16


def _node_prep_body(x_ref, wnT_ref, bn_ref, wmiT_ref, wmjT_ref,
                    xh_ref, a_ref, b_ref):
    xh = jnp.maximum(x_ref[...] @ wnT_ref[...] + bn_ref[...], 0.0)
    xh_ref[...] = xh
    a_ref[...] = xh @ wmiT_ref[...]
    b_ref[...] = xh @ wmjT_ref[...]


def _edge_prep_body(ea_ref, weT_ref, be_ref, wmeT_ref, bM_ref, c_ref):
    eh = jnp.maximum(ea_ref[...] @ weT_ref[...] + be_ref[...], 0.0)
    c_ref[...] = eh @ wmeT_ref[...] + bM_ref[...]


def _sc_msg_body(a_hbm, b_hbm, c_hbm, dst_hbm, src_hbm, zs_hbm, zc_hbm,
                 s_out, cnt_out,
                 dst_v, src_v, dsts_v, a_v, b_v, c_v, m_v, ones_v,
                 s_sh, cnt_sh, sem_ld, sem_g, sem_s, sem_c):
    cid = lax.axis_index("c")
    tid = lax.axis_index("s")
    w = cid * NTILE + tid

    pltpu.sync_copy(zs_hbm.at[pl.ds(tid * ROWS, ROWS)],
                    s_sh.at[pl.ds(tid * ROWS, ROWS)])
    pltpu.sync_copy(zc_hbm.at[pl.ds(tid * ROWS, ROWS)],
                    cnt_sh.at[pl.ds(tid * ROWS, ROWS)])

    def _fill(i, _):
        ones_v[i, :] = jnp.ones((16,), jnp.float32)
        return 0
    lax.fori_loop(0, CHUNK, _fill, 0)

    plsc.subcore_barrier()

    def issue_ld(q, b):
        base = (w + q * NW) * CHUNK
        pltpu.async_copy(dst_hbm.at[pl.ds(base, CHUNK)], dst_v.at[b], sem_ld.at[b])
        pltpu.async_copy(src_hbm.at[pl.ds(base, CHUNK)], src_v.at[b], sem_ld.at[b])
        pltpu.async_copy(c_hbm.at[pl.ds(base, CHUNK)], c_v.at[b], sem_ld.at[b])

    def wait_ld(b):
        pltpu.make_async_copy(dst_hbm.at[pl.ds(0, CHUNK)], dst_v.at[b],
                              sem_ld.at[b]).wait()
        pltpu.make_async_copy(src_hbm.at[pl.ds(0, CHUNK)], src_v.at[b],
                              sem_ld.at[b]).wait()
        pltpu.make_async_copy(c_hbm.at[pl.ds(0, CHUNK)], c_v.at[b],
                              sem_ld.at[b]).wait()

    def issue_gather(b):
        pltpu.async_copy(a_hbm.at[dst_v.at[b]], a_v.at[b], sem_g.at[b])
        pltpu.async_copy(b_hbm.at[src_v.at[b]], b_v.at[b], sem_g.at[b])

    def wait_gather(b):
        pltpu.make_async_copy(a_hbm.at[dst_v.at[b]], a_v.at[b],
                              sem_g.at[b]).wait()
        pltpu.make_async_copy(b_hbm.at[src_v.at[b]], b_v.at[b],
                              sem_g.at[b]).wait()

    def issue_scatter(b):
        pltpu.async_copy(m_v.at[b], s_sh.at[dsts_v.at[b]], sem_s.at[b], add=True)
        pltpu.async_copy(ones_v, cnt_sh.at[dsts_v.at[b]], sem_c.at[b], add=True)

    def wait_scatter(b):
        pltpu.make_async_copy(m_v.at[b], s_sh.at[dsts_v.at[b]],
                              sem_s.at[b]).wait()
        pltpu.make_async_copy(ones_v, cnt_sh.at[dsts_v.at[b]],
                              sem_c.at[b]).wait()

    def compute(b):
        def rows(r, _):
            for u in range(4):
                for j in range(_LANES):
                    sl = pl.ds(j * 16, 16)
                    m_v[b, r * 4 + u, sl] = jnp.maximum(
                        a_v[b, r * 4 + u, sl] + b_v[b, r * 4 + u, sl]
                        + c_v[b, r * 4 + u, sl], 0.0)
            return 0
        lax.fori_loop(0, CHUNK // 4, rows, 0)
        for j in range(CHUNK // 16):
            sl = pl.ds(j * 16, 16)
            dsts_v[b, sl] = dst_v[b, sl]

    NMAIN = NCHUNKS // NW

    issue_ld(0, 0)
    issue_ld(1, 1)
    wait_ld(0)
    issue_gather(0)

    def body(i, _):
        for b in (0, 1):
            q = i * 2 + b
            wait_gather(b)

            @pl.when(q >= 2)
            def _():
                wait_scatter(b)
            compute(b)
            issue_scatter(b)

            @pl.when(q + 2 < NMAIN)
            def _():
                issue_ld(q + 2, b)
            ob = 1 - b

            @pl.when(q + 1 < NMAIN)
            def _():
                wait_ld(ob)
                issue_gather(ob)
        return 0

    lax.fori_loop(0, NMAIN // 2, body, 0)
    wait_scatter(0)
    wait_scatter(1)

    @pl.when(w < NCHUNKS - NMAIN * NW)
    def _():
        issue_ld(NMAIN, 0)
        wait_ld(0)
        issue_gather(0)
        wait_gather(0)
        compute(0)
        issue_scatter(0)
        wait_scatter(0)

    plsc.subcore_barrier()

    pltpu.sync_copy(s_sh.at[pl.ds(tid * ROWS, ROWS)],
                    s_out.at[pl.ds(cid * NPAD + tid * ROWS, ROWS)])
    pltpu.sync_copy(cnt_sh.at[pl.ds(tid * ROWS, ROWS)],
                    cnt_out.at[pl.ds(cid * NPAD + tid * ROWS, ROWS)])


def _update_pool_body(xh_ref, s_ref, cnt_ref, bv_ref,
                      wu1T_ref, wu2T_ref, bU_ref, wrT_ref, br_ref, out_ref):
    s = s_ref[:N, :] + s_ref[NPAD:NPAD + N, :]
    cnt = cnt_ref[:N, 0:1] + cnt_ref[NPAD:NPAD + N, 0:1]
    aggr = s / jnp.maximum(cnt, 1.0)
    upd = jnp.maximum(
        xh_ref[...] @ wu1T_ref[...] + aggr @ wu2T_ref[...] + bU_ref[...], 0.0)
    gids = lax.broadcasted_iota(jnp.int32, (N, G), 1)
    oh = (bv_ref[...] == gids).astype(jnp.float32)
    ps = lax.dot_general(oh, upd, (((0,), (0,)), ((), ())))
    pc = jnp.sum(oh, axis=0)[:, None]
    pooled = ps / jnp.maximum(pc, 1.0)
    out_ref[...] = pooled @ wrT_ref[...] + br_ref[...]


def _full(shape):
    return pl.BlockSpec(shape, lambda *a: tuple(0 for _ in shape))


def kernel(x, edge_index, edge_attr, batch_vec, Wn, bn, We, be, WM, bM,
           WU, bU, Wr, br):
    f32 = jnp.float32
    WnT = Wn.T
    WMiT = WM[:, :H].T
    WMjT = WM[:, H:2 * H].T
    WMeT = WM[:, 2 * H:].T
    WU1T = WU[:, :H].T
    WU2T = WU[:, H:].T
    WrT = Wr.T
    bn2 = bn.reshape(1, H)
    be2 = be.reshape(1, H)
    bM2 = bM.reshape(1, H)
    bU2 = bU.reshape(1, H)
    br2 = br.reshape(1, OUT)
    src = edge_index[0]
    dst = edge_index[1]
    bv2 = batch_vec.reshape(N, 1)

    xh, a_nodes, b_nodes = pl.pallas_call(
        _node_prep_body,
        out_shape=[jax.ShapeDtypeStruct((N, H), f32)] * 3,
        in_specs=[_full((N, DIN)), _full((DIN, H)), _full((1, H)),
                  _full((H, H)), _full((H, H))],
        out_specs=[_full((N, H))] * 3,
    )(x, WnT, bn2, WMiT, WMjT)

    c_edges = pl.pallas_call(
        _edge_prep_body,
        grid=(E // EB,),
        out_shape=jax.ShapeDtypeStruct((E, H), f32),
        in_specs=[pl.BlockSpec((EB, DE), lambda i: (i, 0)),
                  _full((DE, H)), _full((1, H)), _full((H, H)),
                  _full((1, H))],
        out_specs=pl.BlockSpec((EB, H), lambda i: (i, 0)),
    )(edge_attr, We.T, be2, WMeT, bM2)

    mesh = plsc.VectorSubcoreMesh(core_axis_name="c", subcore_axis_name="s",
                                  num_cores=NSC, num_subcores=NTILE)
    sc_call = pl.kernel(
        _sc_msg_body,
        out_type=[jax.ShapeDtypeStruct((NSC * NPAD, H), f32),
                  jax.ShapeDtypeStruct((NSC * NPAD, 16), f32)],
        mesh=mesh,
        compiler_params=pltpu.CompilerParams(use_tc_tiling_on_sc=False),
        scratch_types=[
            pltpu.VMEM((2, CHUNK), jnp.int32),
            pltpu.VMEM((2, CHUNK), jnp.int32),
            pltpu.VMEM((2, CHUNK), jnp.int32),
            pltpu.VMEM((2, CHUNK, H), f32),
            pltpu.VMEM((2, CHUNK, H), f32),
            pltpu.VMEM((2, CHUNK, H), f32),
            pltpu.VMEM((2, CHUNK, H), f32),
            pltpu.VMEM((CHUNK, 16), f32),
            pltpu.VMEM_SHARED((NPAD, H), f32),
            pltpu.VMEM_SHARED((NPAD, 16), f32),
            pltpu.SemaphoreType.DMA((2,)),
            pltpu.SemaphoreType.DMA((2,)),
            pltpu.SemaphoreType.DMA((2,)),
            pltpu.SemaphoreType.DMA((2,)),
        ],
    )
    zs = jnp.zeros((NPAD, H), f32)
    zc = jnp.zeros((NPAD, 16), f32)
    s_par, cnt_par = sc_call(a_nodes, b_nodes, c_edges, dst, src, zs, zc)

    out = pl.pallas_call(
        _update_pool_body,
        out_shape=jax.ShapeDtypeStruct((G, OUT), f32),
        in_specs=[_full((N, H)), _full((NSC * NPAD, H)), _full((NSC * NPAD, 16)),
                  _full((N, 1)), _full((H, H)), _full((H, H)), _full((1, H)),
                  _full((H, OUT)), _full((1, OUT))],
        out_specs=_full((G, OUT)),
    )(xh, s_par, cnt_par, bv2, WU1T, WU2T, bU2, WrT, br2)
    return out


# speedup vs baseline: 4.8842x; 1.0157x over previous
import jax
import jax.numpy as jnp
from jax import lax
from jax.experimental import pallas as pl
from jax.experimental.pallas import tpu as pltpu
from jax.experimental.pallas import tpu_sc as plsc

N = 10000
E = 320000
E2 = E // 2
DIN = 128
DE = 16
H = 64
G = 16
OUT = 1

NSC = 2
NTILE = 16
NW = NSC * NTILE
CHUNK = 128
CROWS = CHUNK // 2
NCHUNKS = E2 // CROWS
NMAIN = NCHUNKS // NW
NPAD = 10240
ROWS = NPAD // NTILE
EB2 = 1280

_LANES = H // 16


def _node_prep_body(x_ref, wnT_ref, bn_ref, wmiT_ref, wmjT_ref,
                    xh_ref, a_ref, b_ref):
    xh = jnp.maximum(x_ref[...] @ wnT_ref[...] + bn_ref[...], 0.0)
    xh_ref[...] = xh
    a_ref[...] = xh @ wmiT_ref[...]
    b_ref[...] = xh @ wmjT_ref[...]


def _edge_prep_body(ea1_ref, ea2_ref, weT_ref, be_ref, wmeT_ref, bM_ref,
                    c_ref):
    eh1 = jnp.maximum(ea1_ref[...] @ weT_ref[...] + be_ref[...], 0.0)
    eh2 = jnp.maximum(ea2_ref[...] @ weT_ref[...] + be_ref[...], 0.0)
    c_ref[:, :H] = eh1 @ wmeT_ref[...] + bM_ref[...]
    c_ref[:, H:] = eh2 @ wmeT_ref[...] + bM_ref[...]


def _sc_msg_body(a_hbm, b_hbm, c_hbm, dst_hbm, src_hbm,
                 s_out, cnt_out,
                 dst0, dst1, src0, src1, dsc0, dsc1,
                 td0, td1, ts0, ts1, cv0, cv1, mv0, mv1, cnt_t,
                 s_sh, sem_ld, sem_g, sem_s):
    cid = lax.axis_index("c")
    tid = lax.axis_index("s")
    w = cid * NTILE + tid
    dst_b = (dst0, dst1)
    src_b = (src0, src1)
    dsc_b = (dsc0, dsc1)
    td_b = (td0, td1)
    ts_b = (ts0, ts1)
    cv_b = (cv0, cv1)
    mv_b = (mv0, mv1)

    zrow = jnp.zeros((16,), jnp.float32)
    ones16 = jnp.ones((16,), jnp.float32)

    def _zfill(e, _):
        for j in range(_LANES):
            mv0[e, pl.ds(j * 16, 16)] = zrow
        return 0
    lax.fori_loop(0, CHUNK, _zfill, 0)
    for k in range(ROWS // CHUNK):
        pltpu.sync_copy(mv0, s_sh.at[pl.ds(tid * ROWS + k * CHUNK, CHUNK)])

    def _czfill(g, _):
        cnt_t[pl.ds(g * 16, 16)] = zrow
        return 0
    lax.fori_loop(0, NPAD // 16, _czfill, 0)

    plsc.subcore_barrier()

    def issue_ld(q, b):
        ch = w + q * NW
        pltpu.async_copy(dst_hbm.at[pl.ds(ch * CHUNK, CHUNK)],
                         dst_b[b], sem_ld.at[b])
        pltpu.async_copy(src_hbm.at[pl.ds(ch * CHUNK, CHUNK)],
                         src_b[b], sem_ld.at[b])
        pltpu.async_copy(c_hbm.at[pl.ds(ch * CROWS, CROWS)], cv_b[b],
                         sem_ld.at[b])

    def wait_ld(b):
        pltpu.make_async_copy(dst_hbm.at[pl.ds(0, CHUNK)], dst_b[b],
                              sem_ld.at[b]).wait()
        pltpu.make_async_copy(src_hbm.at[pl.ds(0, CHUNK)], src_b[b],
                              sem_ld.at[b]).wait()
        pltpu.make_async_copy(c_hbm.at[pl.ds(0, CROWS)], cv_b[b],
                              sem_ld.at[b]).wait()

    def issue_gather(b):
        pltpu.async_copy(a_hbm.at[dst_b[b]], td_b[b], sem_g.at[b])
        pltpu.async_copy(b_hbm.at[src_b[b]], ts_b[b], sem_g.at[b])

    def wait_gather(b):
        pltpu.make_async_copy(a_hbm.at[dst_b[b]], td_b[b], sem_g.at[b]).wait()
        pltpu.make_async_copy(b_hbm.at[src_b[b]], ts_b[b], sem_g.at[b]).wait()

    def issue_scatter(b):
        pltpu.async_copy(mv_b[b], s_sh.at[dsc_b[b]], sem_s.at[b], add=True)

    def wait_scatter(b):
        pltpu.make_async_copy(mv_b[b], s_sh.at[dsc_b[b]], sem_s.at[b]).wait()

    def compute(b):
        td, ts, cv, mv = td_b[b], ts_b[b], cv_b[b], mv_b[b]

        def rows(r, _):
            for j in range(_LANES):
                sl = pl.ds(j * 16, 16)
                sh = pl.ds(64 + j * 16, 16)
                mv[r, sl] = jnp.maximum(
                    td[r, sl] + ts[r, sl] + cv[r, sl], 0.0)
                mv[CROWS + r, sl] = jnp.maximum(
                    td[CROWS + r, sl] + ts[CROWS + r, sl] + cv[r, sh], 0.0)
            return 0
        lax.fori_loop(0, CROWS, rows, 0)
        dst, dsc = dst_b[b], dsc_b[b]
        for j in range(CHUNK // 16):
            sl = pl.ds(j * 16, 16)
            idx = dst[sl]
            dsc[sl] = idx
            plsc.addupdate_scatter(cnt_t, [idx], ones16)

    issue_ld(0, 0)
    issue_ld(1, 1)
    wait_ld(0)
    issue_gather(0)

    def body(i, _):
        for b in (0, 1):
            q = i * 2 + b
            wait_gather(b)

            @pl.when(q >= 2)
            def _():
                wait_scatter(b)
            compute(b)
            issue_scatter(b)

            @pl.when(q + 2 < NMAIN)
            def _():
                issue_ld(q + 2, b)
            ob = 1 - b

            @pl.when(q + 1 < NMAIN)
            def _():
                wait_ld(ob)
                issue_gather(ob)
        return 0

    lax.fori_loop(0, NMAIN // 2, body, 0)
    wait_scatter(0)
    wait_scatter(1)

    @pl.when(w < NCHUNKS - NMAIN * NW)
    def _():
        issue_ld(NMAIN, 0)
        wait_ld(0)
        issue_gather(0)
        wait_gather(0)
        compute(0)
        issue_scatter(0)
        wait_scatter(0)

    plsc.subcore_barrier()

    pltpu.sync_copy(s_sh.at[pl.ds(tid * ROWS, ROWS)],
                    s_out.at[pl.ds(cid * NPAD + tid * ROWS, ROWS)])
    pltpu.sync_copy(cnt_t, cnt_out.at[pl.ds(w * NPAD, NPAD)])


def _update_pool_body(xh_ref, s_ref, cnt_ref, bv_ref,
                      wu1T_ref, wu2T_ref, bU_ref, wrT_ref, br_ref, out_ref):
    s = s_ref[:N, :] + s_ref[NPAD:NPAD + N, :]
    cnt = jnp.sum(cnt_ref[:N, :], axis=1, keepdims=True)
    aggr = s / jnp.maximum(cnt, 1.0)
    upd = jnp.maximum(
        xh_ref[...] @ wu1T_ref[...] + aggr @ wu2T_ref[...] + bU_ref[...], 0.0)
    gids = lax.broadcasted_iota(jnp.int32, (N, G), 1)
    oh = (bv_ref[...] == gids).astype(jnp.float32)
    ps = lax.dot_general(oh, upd, (((0,), (0,)), ((), ())))
    pc = jnp.sum(oh, axis=0)[:, None]
    pooled = ps / jnp.maximum(pc, 1.0)
    out_ref[...] = pooled @ wrT_ref[...] + br_ref[...]


def _full(shape):
    return pl.BlockSpec(shape, lambda *a: tuple(0 for _ in shape))


def kernel(x, edge_index, edge_attr, batch_vec, Wn, bn, We, be, WM, bM,
           WU, bU, Wr, br):
    f32 = jnp.float32
    WnT = Wn.T
    WMiT = WM[:, :H].T
    WMjT = WM[:, H:2 * H].T
    WMeT = WM[:, 2 * H:].T
    WU1T = WU[:, :H].T
    WU2T = WU[:, H:].T
    WrT = Wr.T
    bn2 = bn.reshape(1, H)
    be2 = be.reshape(1, H)
    bM2 = bM.reshape(1, H)
    bU2 = bU.reshape(1, H)
    br2 = br.reshape(1, OUT)
    src = jnp.concatenate([edge_index[0, :E2].reshape(NCHUNKS, CROWS),
                           edge_index[0, E2:].reshape(NCHUNKS, CROWS)],
                          axis=1).reshape(E)
    dst = jnp.concatenate([edge_index[1, :E2].reshape(NCHUNKS, CROWS),
                           edge_index[1, E2:].reshape(NCHUNKS, CROWS)],
                          axis=1).reshape(E)
    bv2 = batch_vec.reshape(N, 1)

    xh, a_nodes, b_nodes = pl.pallas_call(
        _node_prep_body,
        out_shape=[jax.ShapeDtypeStruct((N, H), f32)] * 3,
        in_specs=[_full((N, DIN)), _full((DIN, H)), _full((1, H)),
                  _full((H, H)), _full((H, H))],
        out_specs=[_full((N, H))] * 3,
    )(x, WnT, bn2, WMiT, WMjT)

    c_edges = pl.pallas_call(
        _edge_prep_body,
        grid=(E2 // EB2,),
        out_shape=jax.ShapeDtypeStruct((E2, 2 * H), f32),
        in_specs=[pl.BlockSpec((EB2, DE), lambda i: (i, 0)),
                  pl.BlockSpec((EB2, DE), lambda i: (i + E2 // EB2, 0)),
                  _full((DE, H)), _full((1, H)), _full((H, H)),
                  _full((1, H))],
        out_specs=pl.BlockSpec((EB2, 2 * H), lambda i: (i, 0)),
    )(edge_attr, edge_attr, We.T, be2, WMeT, bM2)

    mesh = plsc.VectorSubcoreMesh(core_axis_name="c", subcore_axis_name="s",
                                  num_cores=NSC, num_subcores=NTILE)
    sc_call = pl.kernel(
        _sc_msg_body,
        out_type=[jax.ShapeDtypeStruct((NSC * NPAD, H), f32),
                  jax.ShapeDtypeStruct((NW * NPAD,), f32)],
        mesh=mesh,
        compiler_params=pltpu.CompilerParams(use_tc_tiling_on_sc=False,
                                             needs_layout_passes=False),
        scratch_types=[
            pltpu.VMEM((CHUNK,), jnp.int32),
            pltpu.VMEM((CHUNK,), jnp.int32),
            pltpu.VMEM((CHUNK,), jnp.int32),
            pltpu.VMEM((CHUNK,), jnp.int32),
            pltpu.VMEM((CHUNK,), jnp.int32),
            pltpu.VMEM((CHUNK,), jnp.int32),
            pltpu.VMEM((CHUNK, H), f32),
            pltpu.VMEM((CHUNK, H), f32),
            pltpu.VMEM((CHUNK, H), f32),
            pltpu.VMEM((CHUNK, H), f32),
            pltpu.VMEM((CROWS, 2 * H), f32),
            pltpu.VMEM((CROWS, 2 * H), f32),
            pltpu.VMEM((CHUNK, H), f32),
            pltpu.VMEM((CHUNK, H), f32),
            pltpu.VMEM((NPAD,), f32),
            pltpu.VMEM_SHARED((NPAD, H), f32),
            pltpu.SemaphoreType.DMA((2,)),
            pltpu.SemaphoreType.DMA((2,)),
            pltpu.SemaphoreType.DMA((2,)),
        ],
    )
    s_par, cnt_flat = sc_call(a_nodes, b_nodes, c_edges, dst, src)
    cntT = cnt_flat.reshape(NW, NPAD).T

    out = pl.pallas_call(
        _update_pool_body,
        out_shape=jax.ShapeDtypeStruct((G, OUT), f32),
        in_specs=[_full((N, H)), _full((NSC * NPAD, H)),
                  _full((NPAD, NW)),
                  _full((N, 1)), _full((H, H)), _full((H, H)), _full((1, H)),
                  _full((H, OUT)), _full((1, OUT))],
        out_specs=_full((G, OUT)),
    )(xh, s_par, cntT, bv2, WU1T, WU2T, bU2, WrT, br2)
    return out


# speedup vs baseline: 4.8853x; 1.0002x over previous
import jax
import jax.numpy as jnp
from jax import lax
from jax.experimental import pallas as pl
from jax.experimental.pallas import tpu as pltpu
from jax.experimental.pallas import tpu_sc as plsc

N = 10000
E = 320000
E2 = E // 2
DIN = 128
DE = 16
H = 64
G = 16
OUT = 1

NSC = 2
NTILE = 16
NW = NSC * NTILE
CHUNK = 128
CROWS = CHUNK // 2
NCHUNKS = E2 // CROWS
NMAIN = NCHUNKS // NW
NPAD = 10240
ROWS = NPAD // NTILE
EB2 = 1280

_LANES = H // 16


def _node_prep_body(x_ref, wnT_ref, bn_ref, wmiT_ref, wmjT_ref,
                    xh_ref, a_ref, b_ref):
    xh = jnp.maximum(x_ref[...] @ wnT_ref[...] + bn_ref[...], 0.0)
    xh_ref[...] = xh
    a_ref[...] = xh @ wmiT_ref[...]
    b_ref[...] = xh @ wmjT_ref[...]


def _edge_prep_body(ea1_ref, ea2_ref, weT_ref, be_ref, wmeT_ref, bM_ref,
                    c_ref):
    eh1 = jnp.maximum(ea1_ref[...] @ weT_ref[...] + be_ref[...], 0.0)
    eh2 = jnp.maximum(ea2_ref[...] @ weT_ref[...] + be_ref[...], 0.0)
    c_ref[:, :H] = eh1 @ wmeT_ref[...] + bM_ref[...]
    c_ref[:, H:] = eh2 @ wmeT_ref[...] + bM_ref[...]


def _sc_msg_body(a_hbm, b_hbm, c_hbm, dst_hbm, src_hbm,
                 s_out, cnt_out,
                 dst0, dst1, src0, src1, dsc0, dsc1,
                 td0, td1, ts0, ts1, cv0, cv1, mv0, mv1, ones_v, zc_v,
                 s_sh, cnt_sh, sem_ld, sem_g, sem_s, sem_c):
    cid = lax.axis_index("c")
    tid = lax.axis_index("s")
    w = cid * NTILE + tid
    dst_b = (dst0, dst1)
    src_b = (src0, src1)
    dsc_b = (dsc0, dsc1)
    td_b = (td0, td1)
    ts_b = (ts0, ts1)
    cv_b = (cv0, cv1)
    mv_b = (mv0, mv1)

    zrow = jnp.zeros((16,), jnp.float32)
    ones16 = jnp.ones((16,), jnp.float32)

    def _zfill(e, _):
        for j in range(_LANES):
            mv0[e, pl.ds(j * 16, 16)] = zrow
        zc_v[e, pl.ds(0, 16)] = zrow
        ones_v[e, pl.ds(0, 16)] = ones16
        return 0
    lax.fori_loop(0, CHUNK, _zfill, 0)
    for k in range(ROWS // CHUNK):
        pltpu.sync_copy(mv0, s_sh.at[pl.ds(tid * ROWS + k * CHUNK, CHUNK)])
        pltpu.sync_copy(zc_v, cnt_sh.at[pl.ds(tid * ROWS + k * CHUNK, CHUNK)])

    plsc.subcore_barrier()

    def issue_ld(q, b):
        ch = w + q * NW
        pltpu.async_copy(dst_hbm.at[pl.ds(ch * CHUNK, CHUNK)],
                         dst_b[b], sem_ld.at[b])
        pltpu.async_copy(src_hbm.at[pl.ds(ch * CHUNK, CHUNK)],
                         src_b[b], sem_ld.at[b])
        pltpu.async_copy(c_hbm.at[pl.ds(ch * CROWS * 2 * H, CROWS * 2 * H)],
                         cv_b[b], sem_ld.at[b])

    def wait_ld(b):
        pltpu.make_async_copy(dst_hbm.at[pl.ds(0, CHUNK)], dst_b[b],
                              sem_ld.at[b]).wait()
        pltpu.make_async_copy(src_hbm.at[pl.ds(0, CHUNK)], src_b[b],
                              sem_ld.at[b]).wait()
        pltpu.make_async_copy(c_hbm.at[pl.ds(0, CROWS * 2 * H)], cv_b[b],
                              sem_ld.at[b]).wait()

    def issue_gather(b):
        pltpu.async_copy(a_hbm.at[dst_b[b]], td_b[b], sem_g.at[b])
        pltpu.async_copy(b_hbm.at[src_b[b]], ts_b[b], sem_g.at[b])

    def wait_gather(b):
        pltpu.make_async_copy(a_hbm.at[dst_b[b]], td_b[b], sem_g.at[b]).wait()
        pltpu.make_async_copy(b_hbm.at[src_b[b]], ts_b[b], sem_g.at[b]).wait()

    def issue_scatter(b):
        pltpu.async_copy(mv_b[b], s_sh.at[dsc_b[b]], sem_s.at[b], add=True)
        pltpu.async_copy(ones_v, cnt_sh.at[dsc_b[b]], sem_c.at[b], add=True)

    def wait_scatter(b):
        pltpu.make_async_copy(mv_b[b], s_sh.at[dsc_b[b]], sem_s.at[b]).wait()
        pltpu.make_async_copy(ones_v, cnt_sh.at[dsc_b[b]],
                              sem_c.at[b]).wait()

    def compute(b):
        td, ts, cv, mv = td_b[b], ts_b[b], cv_b[b], mv_b[b]

        def rows(r, _):
            for j in range(_LANES):
                sl = pl.ds(j * 16, 16)
                sh = pl.ds(64 + j * 16, 16)
                mv[r, sl] = jnp.maximum(
                    td[r, sl] + ts[r, sl]
                    + cv[pl.ds(r * 2 * H + j * 16, 16)], 0.0)
                mv[CROWS + r, sl] = jnp.maximum(
                    td[CROWS + r, sl] + ts[CROWS + r, sl]
                    + cv[pl.ds(r * 2 * H + H + j * 16, 16)], 0.0)
            return 0
        lax.fori_loop(0, CROWS, rows, 0)
        dst, dsc = dst_b[b], dsc_b[b]
        for j in range(CHUNK // 16):
            sl = pl.ds(j * 16, 16)
            dsc[sl] = dst[sl]

    issue_ld(0, 0)
    issue_ld(1, 1)
    wait_ld(0)
    issue_gather(0)

    def body(i, _):
        for b in (0, 1):
            q = i * 2 + b
            wait_gather(b)

            @pl.when(q >= 2)
            def _():
                wait_scatter(b)
            compute(b)
            issue_scatter(b)

            @pl.when(q + 2 < NMAIN)
            def _():
                issue_ld(q + 2, b)
            ob = 1 - b

            @pl.when(q + 1 < NMAIN)
            def _():
                wait_ld(ob)
                issue_gather(ob)
        return 0

    lax.fori_loop(0, NMAIN // 2, body, 0)
    wait_scatter(0)
    wait_scatter(1)

    @pl.when(w < NCHUNKS - NMAIN * NW)
    def _():
        issue_ld(NMAIN, 0)
        wait_ld(0)
        issue_gather(0)
        wait_gather(0)
        compute(0)
        issue_scatter(0)
        wait_scatter(0)

    plsc.subcore_barrier()

    pltpu.sync_copy(s_sh.at[pl.ds(tid * ROWS, ROWS)],
                    s_out.at[pl.ds(cid * NPAD + tid * ROWS, ROWS)])
    pltpu.sync_copy(cnt_sh.at[pl.ds(tid * ROWS, ROWS)],
                    cnt_out.at[pl.ds(cid * NPAD + tid * ROWS, ROWS)])


def _update_pool_body(xh_ref, s_ref, cnt_ref, bv_ref,
                      wu1T_ref, wu2T_ref, bU_ref, wrT_ref, br_ref, out_ref):
    s = s_ref[:N, :] + s_ref[NPAD:NPAD + N, :]
    cnt = cnt_ref[:N, 0:1] + cnt_ref[NPAD:NPAD + N, 0:1]
    aggr = s / jnp.maximum(cnt, 1.0)
    upd = jnp.maximum(
        xh_ref[...] @ wu1T_ref[...] + aggr @ wu2T_ref[...] + bU_ref[...], 0.0)
    gids = lax.broadcasted_iota(jnp.int32, (N, G), 1)
    oh = (bv_ref[...] == gids).astype(jnp.float32)
    ps = lax.dot_general(oh, upd, (((0,), (0,)), ((), ())))
    pc = jnp.sum(oh, axis=0)[:, None]
    pooled = ps / jnp.maximum(pc, 1.0)
    out_ref[...] = pooled @ wrT_ref[...] + br_ref[...]


def _full(shape):
    return pl.BlockSpec(shape, lambda *a: tuple(0 for _ in shape))


def kernel(x, edge_index, edge_attr, batch_vec, Wn, bn, We, be, WM, bM,
           WU, bU, Wr, br):
    f32 = jnp.float32
    WnT = Wn.T
    WMiT = WM[:, :H].T
    WMjT = WM[:, H:2 * H].T
    WMeT = WM[:, 2 * H:].T
    WU1T = WU[:, :H].T
    WU2T = WU[:, H:].T
    WrT = Wr.T
    bn2 = bn.reshape(1, H)
    be2 = be.reshape(1, H)
    bM2 = bM.reshape(1, H)
    bU2 = bU.reshape(1, H)
    br2 = br.reshape(1, OUT)
    src = jnp.concatenate([edge_index[0, :E2].reshape(NCHUNKS, CROWS),
                           edge_index[0, E2:].reshape(NCHUNKS, CROWS)],
                          axis=1).reshape(E)
    dst = jnp.concatenate([edge_index[1, :E2].reshape(NCHUNKS, CROWS),
                           edge_index[1, E2:].reshape(NCHUNKS, CROWS)],
                          axis=1).reshape(E)
    bv2 = batch_vec.reshape(N, 1)

    xh, a_nodes, b_nodes = pl.pallas_call(
        _node_prep_body,
        out_shape=[jax.ShapeDtypeStruct((N, H), f32)] * 3,
        in_specs=[_full((N, DIN)), _full((DIN, H)), _full((1, H)),
                  _full((H, H)), _full((H, H))],
        out_specs=[_full((N, H))] * 3,
    )(x, WnT, bn2, WMiT, WMjT)

    c_edges = pl.pallas_call(
        _edge_prep_body,
        grid=(E2 // EB2,),
        out_shape=jax.ShapeDtypeStruct((E2, 2 * H), f32),
        in_specs=[pl.BlockSpec((EB2, DE), lambda i: (i, 0)),
                  pl.BlockSpec((EB2, DE), lambda i: (i + E2 // EB2, 0)),
                  _full((DE, H)), _full((1, H)), _full((H, H)),
                  _full((1, H))],
        out_specs=pl.BlockSpec((EB2, 2 * H), lambda i: (i, 0)),
    )(edge_attr, edge_attr, We.T, be2, WMeT, bM2)

    c_flat = c_edges.reshape(E2 * 2 * H)

    mesh = plsc.VectorSubcoreMesh(core_axis_name="c", subcore_axis_name="s",
                                  num_cores=NSC, num_subcores=NTILE)
    sc_call = pl.kernel(
        _sc_msg_body,
        out_type=[jax.ShapeDtypeStruct((NSC * NPAD, H), f32),
                  jax.ShapeDtypeStruct((NSC * NPAD, 16), f32)],
        mesh=mesh,
        compiler_params=pltpu.CompilerParams(use_tc_tiling_on_sc=False),
        scratch_types=[
            pltpu.VMEM((CHUNK,), jnp.int32),
            pltpu.VMEM((CHUNK,), jnp.int32),
            pltpu.VMEM((CHUNK,), jnp.int32),
            pltpu.VMEM((CHUNK,), jnp.int32),
            pltpu.VMEM((CHUNK,), jnp.int32),
            pltpu.VMEM((CHUNK,), jnp.int32),
            pltpu.VMEM((CHUNK, H), f32),
            pltpu.VMEM((CHUNK, H), f32),
            pltpu.VMEM((CHUNK, H), f32),
            pltpu.VMEM((CHUNK, H), f32),
            pltpu.VMEM((CROWS * 2 * H,), f32),
            pltpu.VMEM((CROWS * 2 * H,), f32),
            pltpu.VMEM((CHUNK, H), f32),
            pltpu.VMEM((CHUNK, H), f32),
            pltpu.VMEM((CHUNK, 16), f32),
            pltpu.VMEM((CHUNK, 16), f32),
            pltpu.VMEM_SHARED((NPAD, H), f32),
            pltpu.VMEM_SHARED((NPAD, 16), f32),
            pltpu.SemaphoreType.DMA((2,)),
            pltpu.SemaphoreType.DMA((2,)),
            pltpu.SemaphoreType.DMA((2,)),
            pltpu.SemaphoreType.DMA((2,)),
        ],
    )
    s_par, cnt_par = sc_call(a_nodes, b_nodes, c_flat, dst, src)

    out = pl.pallas_call(
        _update_pool_body,
        out_shape=jax.ShapeDtypeStruct((G, OUT), f32),
        in_specs=[_full((N, H)), _full((NSC * NPAD, H)),
                  _full((NSC * NPAD, 16)),
                  _full((N, 1)), _full((H, H)), _full((H, H)), _full((1, H)),
                  _full((H, OUT)), _full((1, OUT))],
        out_specs=_full((G, OUT)),
    )(xh, s_par, cnt_par, bv2, WU1T, WU2T, bU2, WrT, br2)
    return out
